# baseline (speedup 1.0000x reference)
import functools

import jax
import jax.numpy as jnp
from jax import lax
from jax.experimental import pallas as pl
from jax.experimental.pallas import tpu as pltpu
from jax.experimental.pallas import tpu_sc as plsc


def _pick_chunk(per_worker: int) -> int:
    for c in range(128, 7, -8):
        if per_worker % c == 0:
            return c
    raise ValueError(f"no valid chunk size for per-worker count {per_worker}")


@functools.lru_cache(maxsize=None)
def _build_gather(n_edges: int, n_rows: int, d: int):
    info = plsc.get_sparse_core_info()
    nc, ns = info.num_cores, info.num_subcores
    nw = nc * ns
    assert n_edges % nw == 0, (n_edges, nw)
    ew = n_edges // nw
    budget_words = 2_095_000 - n_rows * d
    c = nbuf = None
    for cand_c, cand_nbuf in ((16, 15), (40, 6), (40, 5), (16, 10), (80, 3),
                              (80, 2), (40, 2), (16, 2), (8, 2)):
        if ew % cand_c:
            continue
        if 16 * (2 * ew + cand_nbuf * cand_c * d) <= budget_words:
            c, nbuf = cand_c, cand_nbuf
            break
    assert c is not None, (ew, n_rows, d)
    n_chunks = ew // c
    n_rounds = -(-n_chunks // nbuf)

    mesh = plsc.VectorSubcoreMesh(core_axis_name="c", subcore_axis_name="s")

    @functools.partial(
        pl.kernel,
        out_type=[
            jax.ShapeDtypeStruct((n_edges, d), jnp.float32),
            jax.ShapeDtypeStruct((n_edges, d), jnp.float32),
        ],
        mesh=mesh,
        scratch_types=[
            pltpu.VMEM((2 * ew,), jnp.int32),
            pltpu.VMEM_SHARED((n_rows, d), jnp.float32),
        ]
        + [pltpu.VMEM((c, d), jnp.float32) for _ in range(nbuf)]
        + [pltpu.SemaphoreType.DMA for _ in range(2 * nbuf)],
    )
    def gather_kernel(table_hbm, vi_hbm, vj_hbm, vi_out, vj_out,
                      idxv, table_sp, *bufs_and_sems):
        rbufs = bufs_and_sems[:nbuf]
        gsems = bufs_and_sems[nbuf:2 * nbuf]
        ssems = bufs_and_sems[2 * nbuf:]
        sid = lax.axis_index("s")
        wid = sid * nc + lax.axis_index("c")
        base_w = wid * ew

        stripe = -(-n_rows // ns)
        stripe += (-stripe) % 8
        n_stagers = -(-n_rows // stripe)
        tail = n_rows - (n_stagers - 1) * stripe

        @pl.when(sid < n_stagers - 1)
        def _stage_full():
            rbase = sid * stripe
            pltpu.sync_copy(table_hbm.at[pl.ds(rbase, stripe)],
                            table_sp.at[pl.ds(rbase, stripe)])

        @pl.when(sid == n_stagers - 1)
        def _stage_tail():
            rbase = (n_stagers - 1) * stripe
            pltpu.sync_copy(table_hbm.at[pl.ds(rbase, tail)],
                            table_sp.at[pl.ds(rbase, tail)])

        pltpu.sync_copy(vi_hbm.at[pl.ds(base_w, ew)], idxv.at[pl.ds(0, ew)])
        pltpu.sync_copy(vj_hbm.at[pl.ds(base_w, ew)], idxv.at[pl.ds(ew, ew)])
        plsc.subcore_barrier()

        def run_stream(idx_off, out_hbm):
            def out_slice(j):
                return out_hbm.at[pl.ds(base_w + j * c, c)]

            def idx_slice(j):
                return idxv.at[pl.ds(idx_off + j * c, c)]

            def round_body(jj, carry):
                j0 = nbuf * jj
                for b in range(nbuf):
                    @pl.when((jj != 0) & (j0 + b < n_chunks))
                    def _drain(b=b, j0=j0):
                        pltpu.make_async_copy(
                            rbufs[b], out_slice(j0 + b - nbuf), ssems[b]
                        ).wait()

                    @pl.when(j0 + b < n_chunks)
                    def _gather(b=b, j0=j0):
                        src = table_sp if b % 3 else table_hbm
                        pltpu.async_copy(
                            src.at[idx_slice(j0 + b)], rbufs[b], gsems[b])
                for b in range(nbuf):
                    @pl.when(j0 + b < n_chunks)
                    def _scatter(b=b, j0=j0):
                        src = table_sp if b % 3 else table_hbm
                        pltpu.make_async_copy(
                            src.at[idx_slice(j0 + b)], rbufs[b], gsems[b]
                        ).wait()
                        pltpu.async_copy(rbufs[b], out_slice(j0 + b), ssems[b])
                return carry

            lax.fori_loop(0, n_rounds, round_body, 0)
            j0 = (n_rounds - 1) * nbuf
            for b in range(nbuf):
                if j0 + b < n_chunks:
                    pltpu.make_async_copy(
                        rbufs[b], out_slice(j0 + b), ssems[b]).wait()

        run_stream(0, vi_out)
        run_stream(ew, vj_out)

    return gather_kernel


def kernel(inputs, selected_edges):
    b, n_rows, d = inputs.shape
    n_edges = selected_edges.shape[0]
    if b == 1:
        vi = selected_edges[:, 1]
        vj = selected_edges[:, 2]
        table = inputs.reshape(n_rows, d)
    else:
        idx = selected_edges[:, 0]
        vi = idx * n_rows + selected_edges[:, 1]
        vj = idx * n_rows + selected_edges[:, 2]
        table = inputs.reshape(b * n_rows, d)
    fn = _build_gather(n_edges, table.shape[0], d)
    return tuple(fn(table, vi, vj))


# speedup vs baseline: 1.3293x; 1.3293x over previous
import functools

import jax
import jax.numpy as jnp
from jax import lax
from jax.experimental import pallas as pl
from jax.experimental.pallas import tpu as pltpu
from jax.experimental.pallas import tpu_sc as plsc


def _pick_chunk(per_worker: int) -> int:
    for c in range(128, 7, -8):
        if per_worker % c == 0:
            return c
    raise ValueError(f"no valid chunk size for per-worker count {per_worker}")


@functools.lru_cache(maxsize=None)
def _build_gather(n_edges: int, n_rows: int, d: int):
    info = plsc.get_sparse_core_info()
    nc, ns = info.num_cores, info.num_subcores
    nw = nc * ns
    assert n_edges % nw == 0, (n_edges, nw)
    ew = n_edges // nw
    budget_words = 2_095_000 - n_rows * d
    c = nbuf = None
    for cand_c, cand_nbuf in ((16, 15), (40, 6), (40, 5), (16, 10), (80, 3),
                              (80, 2), (40, 2), (16, 2), (8, 2)):
        if ew % cand_c:
            continue
        if 16 * (2 * ew + cand_nbuf * cand_c * d) <= budget_words:
            c, nbuf = cand_c, cand_nbuf
            break
    assert c is not None, (ew, n_rows, d)
    n_chunks = ew // c
    n_rounds = -(-n_chunks // nbuf)

    mesh = plsc.VectorSubcoreMesh(core_axis_name="c", subcore_axis_name="s")

    @functools.partial(
        pl.kernel,
        out_type=[
            jax.ShapeDtypeStruct((n_edges, d), jnp.float32),
            jax.ShapeDtypeStruct((n_edges, d), jnp.float32),
        ],
        mesh=mesh,
        scratch_types=[
            pltpu.VMEM((2 * ew,), jnp.int32),
            pltpu.VMEM_SHARED((n_rows, d), jnp.float32),
        ]
        + [pltpu.VMEM((c, d), jnp.float32) for _ in range(nbuf)]
        + [pltpu.SemaphoreType.DMA for _ in range(2 * nbuf + 1)],
    )
    def gather_kernel(table_hbm, vi_hbm, vj_hbm, vi_out, vj_out,
                      idxv, table_sp, *bufs_and_sems):
        rbufs = bufs_and_sems[:nbuf]
        gsems = bufs_and_sems[nbuf:2 * nbuf]
        ssems = bufs_and_sems[2 * nbuf:3 * nbuf]
        psem = bufs_and_sems[3 * nbuf]
        sid = lax.axis_index("s")
        wid = sid * nc + lax.axis_index("c")
        base_w = wid * ew

        stripe = -(-n_rows // ns)
        stripe += (-stripe) % 8
        rbase = pl.multiple_of(jnp.minimum(sid * stripe, n_rows - stripe), 8)
        tcopy = pltpu.async_copy(table_hbm.at[pl.ds(rbase, stripe)],
                                 table_sp.at[pl.ds(rbase, stripe)], psem)
        icopy = pltpu.async_copy(vi_hbm.at[pl.ds(base_w, ew)],
                                 idxv.at[pl.ds(0, ew)], psem)
        jcopy = pltpu.async_copy(vj_hbm.at[pl.ds(base_w, ew)],
                                 idxv.at[pl.ds(ew, ew)], psem)
        tcopy.wait()
        icopy.wait()
        jcopy.wait()
        plsc.subcore_barrier()

        def run_stream(idx_off, out_hbm):
            def out_slice(j):
                return out_hbm.at[pl.ds(base_w + j * c, c)]

            def idx_slice(j):
                return idxv.at[pl.ds(idx_off + j * c, c)]

            def round_body(jj, carry):
                j0 = nbuf * jj
                for b in range(nbuf):
                    @pl.when((jj != 0) & (j0 + b < n_chunks))
                    def _drain(b=b, j0=j0):
                        pltpu.make_async_copy(
                            rbufs[b], out_slice(j0 + b - nbuf), ssems[b]
                        ).wait()

                    @pl.when(j0 + b < n_chunks)
                    def _gather(b=b, j0=j0):
                        pltpu.async_copy(
                            table_sp.at[idx_slice(j0 + b)], rbufs[b], gsems[b])
                for b in range(nbuf):
                    @pl.when(j0 + b < n_chunks)
                    def _scatter(b=b, j0=j0):
                        pltpu.make_async_copy(
                            table_sp.at[idx_slice(j0 + b)], rbufs[b], gsems[b]
                        ).wait()
                        pltpu.async_copy(rbufs[b], out_slice(j0 + b), ssems[b])
                return carry

            lax.fori_loop(0, n_rounds, round_body, 0)
            j0 = (n_rounds - 1) * nbuf
            for b in range(nbuf):
                if j0 + b < n_chunks:
                    pltpu.make_async_copy(
                        rbufs[b], out_slice(j0 + b), ssems[b]).wait()

        run_stream(0, vi_out)
        run_stream(ew, vj_out)

    return gather_kernel


def kernel(inputs, selected_edges):
    b, n_rows, d = inputs.shape
    n_edges = selected_edges.shape[0]
    if b == 1:
        vi = selected_edges[:, 1]
        vj = selected_edges[:, 2]
        table = inputs.reshape(n_rows, d)
    else:
        idx = selected_edges[:, 0]
        vi = idx * n_rows + selected_edges[:, 1]
        vj = idx * n_rows + selected_edges[:, 2]
        table = inputs.reshape(b * n_rows, d)
    fn = _build_gather(n_edges, table.shape[0], d)
    return tuple(fn(table, vi, vj))


# speedup vs baseline: 1.3327x; 1.0026x over previous
import functools

import jax
import jax.numpy as jnp
from jax import lax
from jax.experimental import pallas as pl
from jax.experimental.pallas import tpu as pltpu
from jax.experimental.pallas import tpu_sc as plsc


def _pick_chunk(per_worker: int) -> int:
    for c in range(128, 7, -8):
        if per_worker % c == 0:
            return c
    raise ValueError(f"no valid chunk size for per-worker count {per_worker}")


@functools.lru_cache(maxsize=None)
def _build_gather(n_edges: int, n_rows: int, d: int):
    info = plsc.get_sparse_core_info()
    nc, ns = info.num_cores, info.num_subcores
    nw = nc * ns
    assert n_edges % nw == 0, (n_edges, nw)
    ew = n_edges // nw
    budget_words = 2_095_000 - n_rows * d
    c = nbuf = None
    for cand_c, cand_nbuf in ((16, 15), (40, 6), (40, 5), (16, 10), (80, 3),
                              (80, 2), (40, 2), (16, 2), (8, 2)):
        if ew % cand_c:
            continue
        if 16 * (2 * ew + cand_nbuf * cand_c * d) <= budget_words:
            c, nbuf = cand_c, cand_nbuf
            break
    assert c is not None, (ew, n_rows, d)
    n_chunks = ew // c
    n_rounds = -(-n_chunks // nbuf)

    mesh = plsc.VectorSubcoreMesh(core_axis_name="c", subcore_axis_name="s")

    @functools.partial(
        pl.kernel,
        out_type=[
            jax.ShapeDtypeStruct((n_edges, d), jnp.float32),
            jax.ShapeDtypeStruct((n_edges, d), jnp.float32),
        ],
        mesh=mesh,
        scratch_types=[
            pltpu.VMEM((2 * ew,), jnp.int32),
            pltpu.VMEM_SHARED((n_rows, d), jnp.float32),
        ]
        + [pltpu.VMEM((c, d), jnp.float32) for _ in range(nbuf)]
        + [pltpu.SemaphoreType.DMA for _ in range(2 * nbuf + 1)],
    )
    def gather_kernel(table_hbm, vi_hbm, vj_hbm, vi_out, vj_out,
                      idxv, table_sp, *bufs_and_sems):
        rbufs = bufs_and_sems[:nbuf]
        gsems = bufs_and_sems[nbuf:2 * nbuf]
        ssems = bufs_and_sems[2 * nbuf:3 * nbuf]
        psem = bufs_and_sems[3 * nbuf]
        sid = lax.axis_index("s")
        wid = sid * nc + lax.axis_index("c")
        base_w = wid * ew

        stripe = -(-n_rows // ns)
        stripe += (-stripe) % 8
        rbase = pl.multiple_of(jnp.minimum(sid * stripe, n_rows - stripe), 8)
        tcopy = pltpu.async_copy(table_hbm.at[pl.ds(rbase, stripe)],
                                 table_sp.at[pl.ds(rbase, stripe)], psem)
        icopy = pltpu.async_copy(vi_hbm.at[pl.ds(base_w, ew)],
                                 idxv.at[pl.ds(0, ew)], psem)
        jcopy = pltpu.async_copy(vj_hbm.at[pl.ds(base_w, ew)],
                                 idxv.at[pl.ds(ew, ew)], psem)
        icopy.wait()
        jcopy.wait()
        for b in range(nbuf):
            pltpu.async_copy(
                table_hbm.at[idxv.at[pl.ds(b * c, c)]], rbufs[b], gsems[b])
        for b in range(nbuf):
            pltpu.make_async_copy(
                table_hbm.at[idxv.at[pl.ds(b * c, c)]], rbufs[b], gsems[b]
            ).wait()
            pltpu.async_copy(
                rbufs[b], vi_out.at[pl.ds(base_w + b * c, c)], ssems[b])
        tcopy.wait()
        plsc.subcore_barrier()

        def run_stream(idx_off, out_hbm, first_round):
            def out_slice(j):
                return out_hbm.at[pl.ds(base_w + j * c, c)]

            def idx_slice(j):
                return idxv.at[pl.ds(idx_off + j * c, c)]

            def round_body(jj, carry):
                j0 = nbuf * jj
                for b in range(nbuf):
                    @pl.when((jj != 0) & (j0 + b < n_chunks))
                    def _drain(b=b, j0=j0):
                        pltpu.make_async_copy(
                            rbufs[b], out_slice(j0 + b - nbuf), ssems[b]
                        ).wait()

                    @pl.when(j0 + b < n_chunks)
                    def _gather(b=b, j0=j0):
                        pltpu.async_copy(
                            table_sp.at[idx_slice(j0 + b)], rbufs[b], gsems[b])
                for b in range(nbuf):
                    @pl.when(j0 + b < n_chunks)
                    def _scatter(b=b, j0=j0):
                        pltpu.make_async_copy(
                            table_sp.at[idx_slice(j0 + b)], rbufs[b], gsems[b]
                        ).wait()
                        pltpu.async_copy(rbufs[b], out_slice(j0 + b), ssems[b])
                return carry

            lax.fori_loop(first_round, n_rounds, round_body, 0)
            j0 = (n_rounds - 1) * nbuf
            for b in range(nbuf):
                if j0 + b < n_chunks:
                    pltpu.make_async_copy(
                        rbufs[b], out_slice(j0 + b), ssems[b]).wait()

        run_stream(0, vi_out, 1)
        run_stream(ew, vj_out, 0)

    return gather_kernel


def kernel(inputs, selected_edges):
    b, n_rows, d = inputs.shape
    n_edges = selected_edges.shape[0]
    if b == 1:
        vi = selected_edges[:, 1]
        vj = selected_edges[:, 2]
        table = inputs.reshape(n_rows, d)
    else:
        idx = selected_edges[:, 0]
        vi = idx * n_rows + selected_edges[:, 1]
        vj = idx * n_rows + selected_edges[:, 2]
        table = inputs.reshape(b * n_rows, d)
    fn = _build_gather(n_edges, table.shape[0], d)
    return tuple(fn(table, vi, vj))


# speedup vs baseline: 1.4157x; 1.0623x over previous
import functools

import jax
import jax.numpy as jnp
from jax import lax
from jax.experimental import pallas as pl
from jax.experimental.pallas import tpu as pltpu
from jax.experimental.pallas import tpu_sc as plsc


def _pick_chunk(per_worker: int) -> int:
    for c in range(128, 7, -8):
        if per_worker % c == 0:
            return c
    raise ValueError(f"no valid chunk size for per-worker count {per_worker}")


@functools.lru_cache(maxsize=None)
def _build_gather(n_edges: int, n_rows: int, d: int):
    info = plsc.get_sparse_core_info()
    nc, ns = info.num_cores, info.num_subcores
    nw = nc * ns
    assert n_edges % nw == 0, (n_edges, nw)
    ew = n_edges // nw
    budget_words = 2_095_000 - n_rows * d
    c = nbuf = None
    for cand_c, cand_nbuf in ((16, 15), (40, 6), (40, 5), (16, 10), (80, 3),
                              (80, 2), (40, 2), (16, 2), (8, 2)):
        if ew % cand_c:
            continue
        if 16 * (2 * ew + cand_nbuf * cand_c * d) <= budget_words:
            c, nbuf = cand_c, cand_nbuf
            break
    assert c is not None, (ew, n_rows, d)
    n_chunks = ew // c
    n_rounds = -(-n_chunks // nbuf)

    mesh = plsc.VectorSubcoreMesh(core_axis_name="c", subcore_axis_name="s")

    @functools.partial(
        pl.kernel,
        out_type=[
            jax.ShapeDtypeStruct((n_edges, d), jnp.float32),
            jax.ShapeDtypeStruct((n_edges, d), jnp.float32),
        ],
        mesh=mesh,
        scratch_types=[
            pltpu.VMEM((2 * ew,), jnp.int32),
            pltpu.VMEM_SHARED((n_rows, d), jnp.float32),
        ]
        + [pltpu.VMEM((c, d), jnp.float32) for _ in range(nbuf)]
        + [pltpu.SemaphoreType.DMA for _ in range(2 * nbuf + 1)],
    )
    def gather_kernel(table_hbm, allidx_hbm, vi_out, vj_out,
                      idxv, table_sp, *bufs_and_sems):
        rbufs = bufs_and_sems[:nbuf]
        gsems = bufs_and_sems[nbuf:2 * nbuf]
        ssems = bufs_and_sems[2 * nbuf:3 * nbuf]
        psem = bufs_and_sems[3 * nbuf]
        sid = lax.axis_index("s")
        wid = sid * nc + lax.axis_index("c")
        base_w = wid * ew

        stripe = -(-n_rows // ns)
        stripe += (-stripe) % 8
        rbase = pl.multiple_of(jnp.minimum(sid * stripe, n_rows - stripe), 8)
        tcopy = pltpu.async_copy(table_hbm.at[pl.ds(rbase, stripe)],
                                 table_sp.at[pl.ds(rbase, stripe)], psem)
        icopy = pltpu.async_copy(allidx_hbm.at[pl.ds(base_w, ew)],
                                 idxv.at[pl.ds(0, ew)], psem)
        jcopy = pltpu.async_copy(allidx_hbm.at[pl.ds(n_edges + base_w, ew)],
                                 idxv.at[pl.ds(ew, ew)], psem)
        icopy.wait()
        jcopy.wait()
        for b in range(nbuf):
            pltpu.async_copy(
                table_hbm.at[idxv.at[pl.ds(b * c, c)]], rbufs[b], gsems[b])
        for b in range(nbuf):
            pltpu.make_async_copy(
                table_hbm.at[idxv.at[pl.ds(b * c, c)]], rbufs[b], gsems[b]
            ).wait()
            pltpu.async_copy(
                rbufs[b], vi_out.at[pl.ds(base_w + b * c, c)], ssems[b])
        tcopy.wait()
        plsc.subcore_barrier()

        def run_stream(idx_off, out_hbm, first_round):
            def out_slice(j):
                return out_hbm.at[pl.ds(base_w + j * c, c)]

            def idx_slice(j):
                return idxv.at[pl.ds(idx_off + j * c, c)]

            def round_body(jj, carry):
                j0 = nbuf * jj
                for b in range(nbuf):
                    @pl.when((jj != 0) & (j0 + b < n_chunks))
                    def _drain(b=b, j0=j0):
                        pltpu.make_async_copy(
                            rbufs[b], out_slice(j0 + b - nbuf), ssems[b]
                        ).wait()

                    @pl.when(j0 + b < n_chunks)
                    def _gather(b=b, j0=j0):
                        pltpu.async_copy(
                            table_sp.at[idx_slice(j0 + b)], rbufs[b], gsems[b])
                for b in range(nbuf):
                    @pl.when(j0 + b < n_chunks)
                    def _scatter(b=b, j0=j0):
                        pltpu.make_async_copy(
                            table_sp.at[idx_slice(j0 + b)], rbufs[b], gsems[b]
                        ).wait()
                        pltpu.async_copy(rbufs[b], out_slice(j0 + b), ssems[b])
                return carry

            lax.fori_loop(first_round, n_rounds, round_body, 0)
            j0 = (n_rounds - 1) * nbuf
            for b in range(nbuf):
                if j0 + b < n_chunks:
                    pltpu.make_async_copy(
                        rbufs[b], out_slice(j0 + b), ssems[b]).wait()

        run_stream(0, vi_out, 1)
        run_stream(ew, vj_out, 0)

    return gather_kernel


def kernel(inputs, selected_edges):
    b, n_rows, d = inputs.shape
    n_edges = selected_edges.shape[0]
    if b == 1:
        idx_all = selected_edges[:, 1:3].T.reshape(-1)
        table = inputs.reshape(n_rows, d)
    else:
        idx = selected_edges[:, 0]
        vi = idx * n_rows + selected_edges[:, 1]
        vj = idx * n_rows + selected_edges[:, 2]
        idx_all = jnp.concatenate([vi, vj])
        table = inputs.reshape(b * n_rows, d)
    fn = _build_gather(n_edges, table.shape[0], d)
    return tuple(fn(table, idx_all))
